# baseline (speedup 1.0000x reference)
import functools

import jax
import jax.numpy as jnp
from jax import lax
from jax.experimental import pallas as pl
from jax.experimental.pallas import tpu as pltpu
from jax.experimental.pallas import tpu_sc as plsc

N_ROWS = 16384
N_COLS = 1000
N_OUT = 200
L = 16

NC = 2
NS = 16
NW = NC * NS

ROWS_PER_W = N_ROWS // NW
BR = 32
NCH = ROWS_PER_W // BR

GROUP_OFFS = tuple(list(range(0, N_OUT - L, L)) + [N_OUT - L])


def _make_sc_kernel():
    mesh = plsc.VectorSubcoreMesh(core_axis_name="c", subcore_axis_name="s")

    @functools.partial(
        pl.kernel,
        mesh=mesh,
        out_type=jax.ShapeDtypeStruct((N_ROWS * N_OUT,), jnp.float32),
        scratch_types=[
            pltpu.VMEM((N_OUT,), jnp.int32),
            pltpu.VMEM((BR * N_COLS,), jnp.float32),
            pltpu.VMEM((BR * N_COLS,), jnp.float32),
            pltpu.VMEM((BR * N_OUT,), jnp.float32),
            pltpu.VMEM((BR * N_OUT,), jnp.float32),
            pltpu.SemaphoreType.DMA,
            pltpu.SemaphoreType.DMA,
            pltpu.SemaphoreType.DMA,
            pltpu.SemaphoreType.DMA,
        ],
        compiler_params=pltpu.CompilerParams(needs_layout_passes=False),
    )
    def sc_gather(x_hbm, mask_hbm, out_hbm, mask_v, x_v0, x_v1, out_v0,
                  out_v1, in_sem0, in_sem1, out_sem0, out_sem1):
        x_vs = (x_v0, x_v1)
        out_vs = (out_v0, out_v1)
        in_sems = (in_sem0, in_sem1)
        out_sems = (out_sem0, out_sem1)
        wid = lax.axis_index("s") * NC + lax.axis_index("c")
        row0 = wid * ROWS_PER_W

        pltpu.sync_copy(mask_hbm, mask_v)
        mvecs = [mask_v[pl.ds(off, L)] for off in GROUP_OFFS]

        def in_copy(c, slot):
            base = row0 + c * BR
            return pltpu.make_async_copy(
                x_hbm.at[pl.ds(base * N_COLS, BR * N_COLS)],
                x_vs[slot], in_sems[slot],
            )

        def out_copy(c, slot):
            base = row0 + c * BR
            return pltpu.make_async_copy(
                out_vs[slot],
                out_hbm.at[pl.ds(base * N_OUT, BR * N_OUT)], out_sems[slot],
            )

        in_copy(0, 0).start()
        in_copy(1, 1).start()

        def pair_body(g, _):
            for slot in range(2):
                c = 2 * g + slot
                in_copy(c, slot).wait()

                @pl.when(c >= 2)
                def _():
                    out_copy(c - 2, slot).wait()

                def row_body(r, _):
                    rbase = r * N_COLS
                    obase = r * N_OUT
                    for i, off in enumerate(GROUP_OFFS):
                        idx = mvecs[i] + rbase
                        vals = plsc.load_gather(x_vs[slot], [idx])
                        out_vs[slot][pl.ds(obase + off, L)] = vals
                    return 0

                lax.fori_loop(0, BR, row_body, 0, unroll=False)

                out_copy(c, slot).start()

                @pl.when(c + 2 < NCH)
                def _():
                    in_copy(c + 2, slot).start()
            return 0

        lax.fori_loop(0, NCH // 2, pair_body, 0, unroll=False)

        out_copy(NCH - 2, 0).wait()
        out_copy(NCH - 1, 1).wait()

    return sc_gather


_sc_gather = _make_sc_kernel()


@jax.jit
def kernel(x, mask):
    out_flat = _sc_gather(x.reshape(-1), mask.astype(jnp.int32))
    return out_flat.reshape(N_ROWS, N_OUT)


# speedup vs baseline: 1.7577x; 1.7577x over previous
import functools

import jax
import jax.numpy as jnp
from jax import lax
from jax.experimental import pallas as pl
from jax.experimental.pallas import tpu as pltpu
from jax.experimental.pallas import tpu_sc as plsc

N_ROWS = 16384
N_COLS = 1000
N_OUT = 200
L = 16

NC = 2
NS = 16
NW = NC * NS

ROWS_PER_W = N_ROWS // NW
BR = 32
NCH = ROWS_PER_W // BR

GROUP_OFFS = tuple(list(range(0, N_OUT - L, L)) + [N_OUT - L])


def _make_sc_kernel():
    mesh = plsc.VectorSubcoreMesh(core_axis_name="c", subcore_axis_name="s")

    @functools.partial(
        pl.kernel,
        mesh=mesh,
        out_type=jax.ShapeDtypeStruct((N_ROWS, N_OUT), jnp.float32),
        scratch_types=[
            pltpu.VMEM((N_OUT,), jnp.int32),
            pltpu.VMEM((BR, N_COLS), jnp.float32),
            pltpu.VMEM((BR, N_COLS), jnp.float32),
            pltpu.VMEM((BR, N_OUT), jnp.float32),
            pltpu.VMEM((BR, N_OUT), jnp.float32),
            pltpu.SemaphoreType.DMA,
            pltpu.SemaphoreType.DMA,
            pltpu.SemaphoreType.DMA,
            pltpu.SemaphoreType.DMA,
        ],
        compiler_params=pltpu.CompilerParams(needs_layout_passes=False),
    )
    def sc_gather(x_hbm, mask_hbm, out_hbm, mask_v, x_v0, x_v1, out_v0,
                  out_v1, in_sem0, in_sem1, out_sem0, out_sem1):
        x_vs = (x_v0, x_v1)
        out_vs = (out_v0, out_v1)
        in_sems = (in_sem0, in_sem1)
        out_sems = (out_sem0, out_sem1)
        wid = lax.axis_index("s") * NC + lax.axis_index("c")
        row0 = wid * ROWS_PER_W

        pltpu.sync_copy(mask_hbm, mask_v)
        mvecs = [mask_v[pl.ds(off, L)] for off in GROUP_OFFS]

        def in_copy(c, slot):
            base = row0 + c * BR
            return pltpu.make_async_copy(
                x_hbm.at[pl.ds(base, BR), :],
                x_vs[slot], in_sems[slot],
            )

        def out_copy(c, slot):
            base = row0 + c * BR
            return pltpu.make_async_copy(
                out_vs[slot],
                out_hbm.at[pl.ds(base, BR), :], out_sems[slot],
            )

        in_copy(0, 0).start()
        in_copy(1, 1).start()

        def pair_body(g, _):
            for slot in range(2):
                c = 2 * g + slot
                in_copy(c, slot).wait()

                @pl.when(c >= 2)
                def _():
                    out_copy(c - 2, slot).wait()

                def row_body(r, _):
                    rvec = jnp.full((L,), r, jnp.int32)
                    for i, off in enumerate(GROUP_OFFS):
                        vals = plsc.load_gather(x_vs[slot], [rvec, mvecs[i]])
                        out_vs[slot][r, pl.ds(off, L)] = vals
                    return 0

                lax.fori_loop(0, BR, row_body, 0, unroll=False)

                out_copy(c, slot).start()

                @pl.when(c + 2 < NCH)
                def _():
                    in_copy(c + 2, slot).start()
            return 0

        lax.fori_loop(0, NCH // 2, pair_body, 0, unroll=False)

        out_copy(NCH - 2, 0).wait()
        out_copy(NCH - 1, 1).wait()

    return sc_gather


_sc_gather = _make_sc_kernel()


@jax.jit
def kernel(x, mask):
    return _sc_gather(x, mask.astype(jnp.int32))


# speedup vs baseline: 7.3313x; 4.1711x over previous
import functools

import jax
import jax.numpy as jnp
from jax import lax
from jax.experimental import pallas as pl
from jax.experimental.pallas import tpu as pltpu
from jax.experimental.pallas import tpu_sc as plsc

N_ROWS = 16384
N_COLS = 1000
N_OUT = 200

NC = 2
NS = 16
NW = NC * NS

W = N_ROWS // NW
SPLIT = 104


def _make_sc_kernel():
    mesh = plsc.VectorSubcoreMesh(core_axis_name="c", subcore_axis_name="s")

    @functools.partial(
        pl.kernel,
        mesh=mesh,
        out_type=jax.ShapeDtypeStruct((N_OUT, N_ROWS), jnp.float32),
        scratch_types=[
            pltpu.VMEM((N_OUT,), jnp.int32),
            pltpu.VMEM((SPLIT, W), jnp.float32),
            pltpu.VMEM((N_OUT - SPLIT, W), jnp.float32),
            pltpu.SemaphoreType.DMA,
            pltpu.SemaphoreType.DMA,
        ],
        compiler_params=pltpu.CompilerParams(needs_layout_passes=False),
    )
    def sc_gather(xt_hbm, mask_hbm, outt_hbm, mask_v, buf0, buf1, sem0, sem1):
        wid = lax.axis_index("s") * NC + lax.axis_index("c")
        c0 = wid * W

        pltpu.sync_copy(mask_hbm, mask_v)

        g0 = pltpu.make_async_copy(
            xt_hbm.at[mask_v.at[pl.ds(0, SPLIT)], pl.ds(c0, W)], buf0, sem0
        )
        g1 = pltpu.make_async_copy(
            xt_hbm.at[mask_v.at[pl.ds(SPLIT, N_OUT - SPLIT)], pl.ds(c0, W)],
            buf1, sem1,
        )
        g0.start()
        g1.start()
        g0.wait()
        g1.wait()

        w0 = pltpu.make_async_copy(
            buf0, outt_hbm.at[pl.ds(0, SPLIT), pl.ds(c0, W)], sem0
        )
        w1 = pltpu.make_async_copy(
            buf1, outt_hbm.at[pl.ds(SPLIT, N_OUT - SPLIT), pl.ds(c0, W)], sem1
        )
        w0.start()
        w1.start()
        w0.wait()
        w1.wait()

    return sc_gather


_sc_gather = _make_sc_kernel()


@jax.jit
def kernel(x, mask):
    out_t = _sc_gather(x.T, mask.astype(jnp.int32))
    return out_t.T
